# baseline (speedup 1.0000x reference)
import functools

import jax
import jax.numpy as jnp
from jax import lax
from jax.experimental import pallas as pl
from jax.experimental.pallas import tpu as pltpu
from jax.experimental.pallas import tpu_sc as plsc

B, N, HW, D = 32, 1000, 16384, 64
NC, NS = 2, 16
NW = NC * NS
NCHUNK, CROWS = 8, 125

_LN2 = 0.6931471805599453
_SQRT2 = 1.4142135623730951


def _ln16(x):
    bits = plsc.bitcast(x, jnp.int32)
    e = (bits >> 23) - 127
    mb = (bits & 0x007FFFFF) | 0x3F800000
    m = plsc.bitcast(mb, jnp.float32)
    big = m > _SQRT2
    m = jnp.where(big, m * 0.5, m)
    ef = e.astype(jnp.float32) + jnp.where(big, 1.0, 0.0)
    s = (m - 1.0) / (m + 1.0)
    z = s * s
    poly = 1.0 + z * (1.0 / 3.0 + z * (1.0 / 5.0 + z * (1.0 / 7.0 + z * (1.0 / 9.0))))
    ln = ef * _LN2 + (2.0 * s) * poly
    return jnp.where(x == 0.0, -jnp.inf, ln)


def _body(table, gidx, maskf, targ, out,
          idx_v, mask_v, pred_a, pred_b, targ_a, targ_b, acc_v,
          sem_ga, sem_gb, sem_ta, sem_tb):
    wid = lax.axis_index("s") * NC + lax.axis_index("c")

    pltpu.sync_copy(gidx.at[wid], idx_v)
    pltpu.sync_copy(maskf.at[wid], mask_v)

    preds = (pred_a, pred_b)
    targs = (targ_a, targ_b)
    gsems = (sem_ga, sem_gb)
    tsems = (sem_ta, sem_tb)

    def start(j):
        s = j % 2
        pltpu.make_async_copy(table.at[idx_v.at[j]], preds[s], gsems[s]).start()
        pltpu.make_async_copy(targ.at[wid, j], targs[s], tsems[s]).start()

    def wait(j):
        s = j % 2
        pltpu.make_async_copy(table.at[idx_v.at[j]], preds[s], gsems[s]).wait()
        pltpu.make_async_copy(targ.at[wid, j], targs[s], tsems[s]).wait()

    start(0)
    acc_l = jnp.zeros((16,), jnp.float32)
    acc_n = jnp.zeros((16,), jnp.float32)
    for j in range(NCHUNK):
        if j + 1 < NCHUNK:
            start(j + 1)
        wait(j)
        pred_v, targ_v = preds[j % 2], targs[j % 2]

        def row(r, carry, pred_v=pred_v, targ_v=targ_v, j=j):
            a_l, a_n = carry
            m = plsc.load_gather(mask_v, [jnp.full((16,), j * CROWS, jnp.int32) + r])
            a_n = a_n + m
            for k in range(D // 16):
                p = pred_v[r, pl.ds(k * 16, 16)]
                t = targ_v[r, pl.ds(k * 16, 16)]
                pos = t == 1.0
                arg = jnp.where(pos, p, 1.0 - p)
                w = jnp.where(pos, 1.5, 1.0) * m
                a_l = a_l + w * _ln16(arg)
            return a_l, a_n

        acc_l, acc_n = lax.fori_loop(0, CROWS, row, (acc_l, acc_n))

    acc_v[0, :] = acc_l
    acc_v[1, :] = acc_n
    pltpu.sync_copy(acc_v, out.at[wid])


@jax.jit
def _mask_loss(table, gidx, maskf, targ):
    mesh = plsc.VectorSubcoreMesh(core_axis_name="c", subcore_axis_name="s")
    parts = pl.kernel(
        _body,
        out_type=jax.ShapeDtypeStruct((NW, 2, 16), jnp.float32),
        mesh=mesh,
        compiler_params=pltpu.CompilerParams(
            needs_layout_passes=False, use_tc_tiling_on_sc=False),
        scratch_types=[
            pltpu.VMEM((NCHUNK, CROWS), jnp.int32),
            pltpu.VMEM((N,), jnp.float32),
            pltpu.VMEM((CROWS, D), jnp.float32),
            pltpu.VMEM((CROWS, D), jnp.float32),
            pltpu.VMEM((CROWS, D), jnp.float32),
            pltpu.VMEM((CROWS, D), jnp.float32),
            pltpu.VMEM((2, 16), jnp.float32),
            pltpu.SemaphoreType.DMA,
            pltpu.SemaphoreType.DMA,
            pltpu.SemaphoreType.DMA,
            pltpu.SemaphoreType.DMA,
        ],
    )(table, gidx, maskf, targ)
    loss = 0.0 - jnp.sum(parts[:, 0, :])
    num = 4.0 * jnp.sum(parts[:, 1, :])
    return jnp.where(num > 0, loss / num, loss)


def kernel(output, mask, ind, target):
    table = output.reshape(B * HW, D)
    gidx = (ind.astype(jnp.int32) + jnp.arange(B, dtype=jnp.int32)[:, None] * HW
            ).reshape(B, NCHUNK, CROWS)
    targ = target.reshape(B, NCHUNK, CROWS, D)
    return _mask_loss(table, gidx, mask.astype(jnp.float32), targ)
